# baseline (speedup 1.0000x reference)
import functools

import jax
import jax.numpy as jnp
from jax import lax
from jax.experimental import pallas as pl
from jax.experimental.pallas import tpu as pltpu
from jax.experimental.pallas import tpu_sc as plsc

B = 16384
V = 1000000
D = 64

_INFO = plsc.get_sparse_core_info()
NC = _INFO.num_cores
NS = _INFO.num_subcores
L = _INFO.num_lanes
NW = NC * NS
BPW = B // NW
CHUNK = 128
NCH = BPW // CHUNK
NDC = D // L


def _mf_body(uid_hbm, iid_hbm, ut_hbm, it_hbm, w_hbm, b_hbm, bu_hbm, bi_hbm,
             out_hbm,
             uidx_v, iidx_v, urows_v, irows_v, buv, biv, wv, bv, pscr, outv,
             sem):
    wid = lax.axis_index("s") * NC + lax.axis_index("c")
    base = wid * NCH

    pltpu.sync_copy(uid_hbm.at[pl.ds(base, NCH)], uidx_v)
    pltpu.sync_copy(iid_hbm.at[pl.ds(base, NCH)], iidx_v)
    pltpu.sync_copy(w_hbm, wv)
    pltpu.sync_copy(b_hbm, bv)

    handles = []
    for j in range(NCH):
        sl = pl.ds(j * CHUNK, CHUNK)
        handles.append(pltpu.async_copy(ut_hbm.at[uidx_v.at[j]],
                                        urows_v.at[sl], sem))
        handles.append(pltpu.async_copy(it_hbm.at[iidx_v.at[j]],
                                        irows_v.at[sl], sem))
        handles.append(pltpu.async_copy(bu_hbm.at[uidx_v.at[j]],
                                        buv.at[sl], sem))
        handles.append(pltpu.async_copy(bi_hbm.at[iidx_v.at[j]],
                                        biv.at[sl], sem))
    for h in handles:
        h.wait()

    wchunks = [wv[pl.ds(c * L, L)] for c in range(NDC)]
    bvec = bv[:]
    lane = lax.iota(jnp.int32, L)

    def group_body(g, carry):
        rowbase = g * L
        for r in range(L):
            row = rowbase + r
            acc = None
            for c in range(NDC):
                u = urows_v[row, pl.ds(c * L, L)]
                v = irows_v[row, pl.ds(c * L, L)]
                t = u * v * wchunks[c]
                acc = t if acc is None else acc + t
            pscr[r, :] = acc
        colsum = None
        for dcol in range(L):
            col = plsc.load_gather(
                pscr, [lane, jnp.full((L,), dcol, jnp.int32)])
            colsum = col if colsum is None else colsum + col
        bu_col = buv[pl.ds(rowbase, L)]
        bi_col = biv[pl.ds(rowbase, L)]
        outv[pl.ds(rowbase, L)] = colsum + bu_col + bi_col + bvec
        return carry

    lax.fori_loop(0, BPW // L, group_body, 0)
    pltpu.sync_copy(outv, out_hbm.at[pl.ds(wid * BPW, BPW)])


@jax.jit
def _mf_sc(uid, iid, user_table, item_table, w, b16, bu_table, bi_table):
    mesh = plsc.VectorSubcoreMesh(core_axis_name="c", subcore_axis_name="s")
    kern = pl.kernel(
        _mf_body,
        mesh=mesh,
        compiler_params=pltpu.CompilerParams(needs_layout_passes=False,
                                             use_tc_tiling_on_sc=False),
        out_type=jax.ShapeDtypeStruct((B,), jnp.float32),
        scratch_types=[
            pltpu.VMEM((NCH, CHUNK), jnp.int32),
            pltpu.VMEM((NCH, CHUNK), jnp.int32),
            pltpu.VMEM((BPW, D), jnp.float32),
            pltpu.VMEM((BPW, D), jnp.float32),
            pltpu.VMEM((BPW,), jnp.float32),
            pltpu.VMEM((BPW,), jnp.float32),
            pltpu.VMEM((D,), jnp.float32),
            pltpu.VMEM((L,), jnp.float32),
            pltpu.VMEM((L, L), jnp.float32),
            pltpu.VMEM((BPW,), jnp.float32),
            pltpu.SemaphoreType.DMA,
        ],
    )
    return kern(uid, iid, user_table, item_table, w, b16, bu_table, bi_table)


def kernel(user_review, item_review, user_id, item_id, item_id_per_review,
           user_id_per_review, user_table, item_table, W, b, bu_table,
           bi_table):
    uid = user_id.reshape(NW * NCH, CHUNK)
    iid = item_id.reshape(NW * NCH, CHUNK)
    w = W.reshape(D)
    b16 = jnp.broadcast_to(b, (L,))
    return _mf_sc(uid, iid, user_table, item_table, w, b16,
                  bu_table.reshape(V + 1), bi_table.reshape(V + 1))


# speedup vs baseline: 1.3986x; 1.3986x over previous
import functools

import jax
import jax.numpy as jnp
from jax import lax
from jax.experimental import pallas as pl
from jax.experimental.pallas import tpu as pltpu
from jax.experimental.pallas import tpu_sc as plsc

B = 16384
V = 1000000
D = 64

_INFO = plsc.get_sparse_core_info()
NC = _INFO.num_cores
NS = _INFO.num_subcores
L = _INFO.num_lanes
NW = NC * NS
BPW = B // NW
CHUNK = 128
NCH = BPW // CHUNK
NDC = D // L


def _mf_body(uid_hbm, iid_hbm, ut_hbm, it_hbm, w_hbm, b_hbm, bu_hbm, bi_hbm,
             out_hbm,
             uidx_v, iidx_v, uw_v, iw_v, buv, biv, wv, bv, pscr, outv,
             sem0, sem1, bsem):
    wid = lax.axis_index("s") * NC + lax.axis_index("c")
    base = wid * NCH

    pltpu.sync_copy(uid_hbm.at[pl.ds(base, NCH)], uidx_v)
    pltpu.sync_copy(iid_hbm.at[pl.ds(base, NCH)], iidx_v)
    pltpu.sync_copy(w_hbm, wv)
    pltpu.sync_copy(b_hbm, bv)

    bias_handles = []
    for j in range(NCH):
        sl = pl.ds(j * CHUNK, CHUNK)
        bias_handles.append(
            pltpu.async_copy(bu_hbm.at[uidx_v.at[j]], buv.at[sl], bsem))
        bias_handles.append(
            pltpu.async_copy(bi_hbm.at[iidx_v.at[j]], biv.at[sl], bsem))

    sems = [sem0, sem1]

    def fire_wave(j):
        jj = j % 2
        sem = sems[jj]

        def fire_group(g, carry):
            p0 = g * L
            u16 = uidx_v[j, pl.ds(p0, L)]
            i16 = iidx_v[j, pl.ds(p0, L)]
            for r in range(L):
                pltpu.async_copy(ut_hbm.at[pl.ds(u16[r], 1)],
                                 uw_v.at[jj, pl.ds(p0 + r, 1)], sem)
                pltpu.async_copy(it_hbm.at[pl.ds(i16[r], 1)],
                                 iw_v.at[jj, pl.ds(p0 + r, 1)], sem)
            return carry

        lax.fori_loop(0, CHUNK // L, fire_group, 0)

    def drain_wave(j):
        jj = j % 2
        sem = sems[jj]
        pltpu.make_async_copy(ut_hbm.at[pl.ds(0, CHUNK)], uw_v.at[jj],
                              sem).wait()
        pltpu.make_async_copy(it_hbm.at[pl.ds(0, CHUNK)], iw_v.at[jj],
                              sem).wait()

    wchunks = [wv[pl.ds(c * L, L)] for c in range(NDC)]
    bvec = bv[:]
    lane = lax.iota(jnp.int32, L)

    def compute_wave(j):
        jj = j % 2
        rows_u = uw_v
        rows_i = iw_v

        def group_body(g, carry):
            rowbase = g * L
            for r in range(L):
                row = rowbase + r
                acc = None
                for c in range(NDC):
                    u = rows_u[jj, row, pl.ds(c * L, L)]
                    v = rows_i[jj, row, pl.ds(c * L, L)]
                    t = u * v * wchunks[c]
                    acc = t if acc is None else acc + t
                pscr[r, :] = acc
            colsum = None
            for dcol in range(L):
                col = plsc.load_gather(
                    pscr, [lane, jnp.full((L,), dcol, jnp.int32)])
                colsum = col if colsum is None else colsum + col
            outv[pl.ds(j * CHUNK + rowbase, L)] = colsum
            return carry

        lax.fori_loop(0, CHUNK // L, group_body, 0)

    fire_wave(0)
    for j in range(NCH):
        if j + 1 < NCH:
            fire_wave(j + 1)
        drain_wave(j)
        compute_wave(j)

    for h in bias_handles:
        h.wait()

    def bias_body(g, carry):
        rowbase = g * L
        bu_col = buv[pl.ds(rowbase, L)]
        bi_col = biv[pl.ds(rowbase, L)]
        outv[pl.ds(rowbase, L)] = (outv[pl.ds(rowbase, L)] + bu_col + bi_col
                                   + bvec)
        return carry

    lax.fori_loop(0, BPW // L, bias_body, 0)
    pltpu.sync_copy(outv, out_hbm.at[pl.ds(wid * BPW, BPW)])


@jax.jit
def _mf_sc(uid, iid, user_table, item_table, w, b16, bu_flat, bi_flat):
    mesh = plsc.VectorSubcoreMesh(core_axis_name="c", subcore_axis_name="s")
    kern = pl.kernel(
        _mf_body,
        mesh=mesh,
        compiler_params=pltpu.CompilerParams(needs_layout_passes=False,
                                             use_tc_tiling_on_sc=True),
        out_type=jax.ShapeDtypeStruct((B,), jnp.float32),
        scratch_types=[
            pltpu.VMEM((NCH, CHUNK), jnp.int32),
            pltpu.VMEM((NCH, CHUNK), jnp.int32),
            pltpu.VMEM((2, CHUNK, D), jnp.float32),
            pltpu.VMEM((2, CHUNK, D), jnp.float32),
            pltpu.VMEM((BPW,), jnp.float32),
            pltpu.VMEM((BPW,), jnp.float32),
            pltpu.VMEM((D,), jnp.float32),
            pltpu.VMEM((L,), jnp.float32),
            pltpu.VMEM((L, L), jnp.float32),
            pltpu.VMEM((BPW,), jnp.float32),
            pltpu.SemaphoreType.DMA,
            pltpu.SemaphoreType.DMA,
            pltpu.SemaphoreType.DMA,
        ],
    )
    return kern(uid, iid, user_table, item_table, w, b16, bu_flat, bi_flat)


def kernel(user_review, item_review, user_id, item_id, item_id_per_review,
           user_id_per_review, user_table, item_table, W, b, bu_table,
           bi_table):
    uid = user_id.reshape(NW * NCH, CHUNK)
    iid = item_id.reshape(NW * NCH, CHUNK)
    w = W.reshape(D)
    b16 = jnp.broadcast_to(b, (L,))
    return _mf_sc(uid, iid, user_table, item_table, w, b16,
                  bu_table.reshape(V + 1), bi_table.reshape(V + 1))


# speedup vs baseline: 1.4006x; 1.0015x over previous
import functools

import jax
import jax.numpy as jnp
from jax import lax
from jax.experimental import pallas as pl
from jax.experimental.pallas import tpu as pltpu
from jax.experimental.pallas import tpu_sc as plsc

B = 16384
V = 1000000
D = 64

_INFO = plsc.get_sparse_core_info()
NC = _INFO.num_cores
NS = _INFO.num_subcores
L = _INFO.num_lanes
NW = NC * NS
BPW = B // NW
CHUNK = 128
NCH = BPW // CHUNK
NDC = D // L


def _mf_body(uid_hbm, iid_hbm, ut_hbm, it_hbm, w_hbm, b_hbm, bu_hbm, bi_hbm,
             out_hbm,
             uidx_v, iidx_v, uw_v, iw_v, buv, biv, wv, bv, pscr, outv,
             sem0, sem1, bsem):
    wid = lax.axis_index("s") * NC + lax.axis_index("c")
    base = wid * NCH

    pltpu.sync_copy(uid_hbm.at[pl.ds(base, NCH)], uidx_v)
    pltpu.sync_copy(iid_hbm.at[pl.ds(base, NCH)], iidx_v)
    pltpu.sync_copy(w_hbm, wv)
    pltpu.sync_copy(b_hbm, bv)

    bias_handles = []
    for j in range(NCH):
        sl = pl.ds(j * CHUNK, CHUNK)
        bias_handles.append(
            pltpu.async_copy(bu_hbm.at[uidx_v.at[j]], buv.at[sl], bsem))
        bias_handles.append(
            pltpu.async_copy(bi_hbm.at[iidx_v.at[j]], biv.at[sl], bsem))

    sems = [sem0, sem1]

    def fire_wave(j):
        jj = j % 2
        sem = sems[jj]

        def fire_group(g, carry):
            p0 = g * L
            u16 = uidx_v[j, pl.ds(p0, L)]
            i16 = iidx_v[j, pl.ds(p0, L)]
            for r in range(L):
                pltpu.async_copy(ut_hbm.at[pl.ds(u16[r], 1)],
                                 uw_v.at[jj, pl.ds(p0 + r, 1)], sem)
                pltpu.async_copy(it_hbm.at[pl.ds(i16[r], 1)],
                                 iw_v.at[jj, pl.ds(p0 + r, 1)], sem)
            return carry

        lax.fori_loop(0, CHUNK // L, fire_group, 0)

    def drain_wave(j):
        jj = j % 2
        sem = sems[jj]
        pltpu.make_async_copy(ut_hbm.at[pl.ds(0, CHUNK)], uw_v.at[jj],
                              sem).wait()
        pltpu.make_async_copy(it_hbm.at[pl.ds(0, CHUNK)], iw_v.at[jj],
                              sem).wait()

    wchunks = [wv[pl.ds(c * L, L)] for c in range(NDC)]
    bvec = bv[:]
    lane = lax.iota(jnp.int32, L)

    def compute_wave(j):
        jj = j % 2
        rows_u = uw_v
        rows_i = iw_v

        def group_body(g, carry):
            rowbase = g * L
            for r in range(L):
                row = rowbase + r
                acc = None
                for c in range(NDC):
                    u = rows_u[jj, row, pl.ds(c * L, L)]
                    v = rows_i[jj, row, pl.ds(c * L, L)]
                    t = u * v * wchunks[c]
                    acc = t if acc is None else acc + t
                pscr[r, :] = acc
            colsum = None
            for dcol in range(L):
                col = plsc.load_gather(
                    pscr, [lane, jnp.full((L,), dcol, jnp.int32)])
                colsum = col if colsum is None else colsum + col
            outv[pl.ds(j * CHUNK + rowbase, L)] = colsum
            return carry

        lax.fori_loop(0, CHUNK // L, group_body, 0)

    fire_wave(0)
    for j in range(NCH):
        if j + 1 < NCH:
            fire_wave(j + 1)
        drain_wave(j)
        compute_wave(j)

    for h in bias_handles:
        h.wait()

    def bias_body(g, carry):
        rowbase = g * L
        bu_col = buv[pl.ds(rowbase, L)]
        bi_col = biv[pl.ds(rowbase, L)]
        outv[pl.ds(rowbase, L)] = (outv[pl.ds(rowbase, L)] + bu_col + bi_col
                                   + bvec)
        return carry

    lax.fori_loop(0, BPW // L, bias_body, 0)
    pltpu.sync_copy(outv, out_hbm.at[pl.ds(wid * BPW, BPW)])


@jax.jit
def _mf_sc(uid, iid, user_table, item_table, w, b16, bu_flat, bi_flat):
    mesh = plsc.VectorSubcoreMesh(core_axis_name="c", subcore_axis_name="s")
    kern = pl.kernel(
        _mf_body,
        mesh=mesh,
        compiler_params=pltpu.CompilerParams(needs_layout_passes=False,
                                             use_tc_tiling_on_sc=True),
        out_type=jax.ShapeDtypeStruct((B,), jnp.float32),
        scratch_types=[
            pltpu.VMEM((NCH, CHUNK), jnp.int32),
            pltpu.VMEM((NCH, CHUNK), jnp.int32),
            pltpu.VMEM((2, CHUNK, D), jnp.float32),
            pltpu.VMEM((2, CHUNK, D), jnp.float32),
            pltpu.VMEM((BPW,), jnp.float32),
            pltpu.VMEM((BPW,), jnp.float32),
            pltpu.VMEM((D,), jnp.float32),
            pltpu.VMEM((L,), jnp.float32),
            pltpu.VMEM((L, L), jnp.float32),
            pltpu.VMEM((BPW,), jnp.float32),
            pltpu.SemaphoreType.DMA,
            pltpu.SemaphoreType.DMA,
            pltpu.SemaphoreType.DMA,
        ],
    )
    return kern(uid, iid, user_table, item_table, w, b16, bu_flat, bi_flat)


def kernel(user_review, item_review, user_id, item_id, item_id_per_review,
           user_id_per_review, user_table, item_table, W, b, bu_table,
           bi_table):
    uid = user_id.reshape(NW * NCH, CHUNK)
    iid = item_id.reshape(NW * NCH, CHUNK)
    w = W.reshape(D)
    b16 = jnp.broadcast_to(b, (L,))
    return _mf_sc(uid, iid, user_table, item_table, w, b16,
                  bu_table[:, 0], bi_table[:, 0])


# speedup vs baseline: 1.4021x; 1.0011x over previous
import jax
import jax.numpy as jnp
from jax import lax
from jax.experimental import pallas as pl
from jax.experimental.pallas import tpu as pltpu
from jax.experimental.pallas import tpu_sc as plsc

B = 16384
V = 1000000
D = 64

_INFO = plsc.get_sparse_core_info()
NC = _INFO.num_cores
NS = _INFO.num_subcores
L = _INFO.num_lanes
NW = NC * NS
BPW = B // NW
CHUNK = 128
NCH = BPW // CHUNK
NDC = D // L


def _bias_body(uid_hbm, iid_hbm, buT_hbm, biT_hbm, b_hbm, out_hbm,
               uidx_v, iidx_v, buv, biv, bv, outv, bsem):
    wid = lax.axis_index("s") * NC + lax.axis_index("c")
    base = wid * NCH

    pltpu.sync_copy(uid_hbm.at[pl.ds(base, NCH)], uidx_v)
    pltpu.sync_copy(iid_hbm.at[pl.ds(base, NCH)], iidx_v)
    pltpu.sync_copy(b_hbm, bv)

    handles = []
    for j in range(NCH):
        sl = pl.ds(j * CHUNK, CHUNK)
        handles.append(
            pltpu.async_copy(buT_hbm.at[0].at[uidx_v.at[j]], buv.at[sl],
                             bsem))
        handles.append(
            pltpu.async_copy(biT_hbm.at[0].at[iidx_v.at[j]], biv.at[sl],
                             bsem))
    for h in handles:
        h.wait()

    bvec = bv[:]

    def body(g, carry):
        rowbase = g * L
        outv[pl.ds(rowbase, L)] = (buv[pl.ds(rowbase, L)]
                                   + biv[pl.ds(rowbase, L)] + bvec)
        return carry

    lax.fori_loop(0, BPW // L, body, 0)
    pltpu.sync_copy(outv, out_hbm.at[pl.ds(wid * BPW, BPW)])


def _mf_body(uid_hbm, iid_hbm, ut_hbm, it_hbm, w_hbm, part_hbm, out_hbm,
             uidx_v, iidx_v, uw_v, iw_v, pav, wv, pscr, outv,
             sem0, sem1):
    wid = lax.axis_index("s") * NC + lax.axis_index("c")
    base = wid * NCH

    pltpu.sync_copy(uid_hbm.at[pl.ds(base, NCH)], uidx_v)
    pltpu.sync_copy(iid_hbm.at[pl.ds(base, NCH)], iidx_v)
    pltpu.sync_copy(w_hbm, wv)
    pltpu.sync_copy(part_hbm.at[pl.ds(wid * BPW, BPW)], pav)

    sems = [sem0, sem1]

    def fire_wave(j):
        jj = j % 2
        sem = sems[jj]

        def fire_group(g, carry):
            p0 = g * L
            u16 = uidx_v[j, pl.ds(p0, L)]
            i16 = iidx_v[j, pl.ds(p0, L)]
            for r in range(L):
                pltpu.async_copy(ut_hbm.at[pl.ds(u16[r], 1)],
                                 uw_v.at[jj, pl.ds(p0 + r, 1)], sem)
                pltpu.async_copy(it_hbm.at[pl.ds(i16[r], 1)],
                                 iw_v.at[jj, pl.ds(p0 + r, 1)], sem)
            return carry

        lax.fori_loop(0, CHUNK // L, fire_group, 0)

    def drain_wave(j):
        jj = j % 2
        sem = sems[jj]
        pltpu.make_async_copy(ut_hbm.at[pl.ds(0, CHUNK)], uw_v.at[jj],
                              sem).wait()
        pltpu.make_async_copy(it_hbm.at[pl.ds(0, CHUNK)], iw_v.at[jj],
                              sem).wait()

    wchunks = [wv[pl.ds(c * L, L)] for c in range(NDC)]
    lane = lax.iota(jnp.int32, L)

    def compute_wave(j):
        jj = j % 2

        def group_body(g, carry):
            rowbase = g * L
            for r in range(L):
                row = rowbase + r
                acc = None
                for c in range(NDC):
                    u = uw_v[jj, row, pl.ds(c * L, L)]
                    v = iw_v[jj, row, pl.ds(c * L, L)]
                    t = u * v * wchunks[c]
                    acc = t if acc is None else acc + t
                pscr[r, :] = acc
            colsum = None
            for dcol in range(L):
                col = plsc.load_gather(
                    pscr, [lane, jnp.full((L,), dcol, jnp.int32)])
                colsum = col if colsum is None else colsum + col
            p0 = j * CHUNK + rowbase
            outv[pl.ds(p0, L)] = colsum + pav[pl.ds(p0, L)]
            return carry

        lax.fori_loop(0, CHUNK // L, group_body, 0)

    fire_wave(0)
    for j in range(NCH):
        if j + 1 < NCH:
            fire_wave(j + 1)
        drain_wave(j)
        compute_wave(j)

    pltpu.sync_copy(outv, out_hbm.at[pl.ds(wid * BPW, BPW)])


@jax.jit
def _mf_sc(uid, iid, user_table, item_table, w, b16, buT, biT):
    mesh = plsc.VectorSubcoreMesh(core_axis_name="c", subcore_axis_name="s")
    bias_kern = pl.kernel(
        _bias_body,
        mesh=mesh,
        compiler_params=pltpu.CompilerParams(needs_layout_passes=False,
                                             use_tc_tiling_on_sc=False),
        out_type=jax.ShapeDtypeStruct((B,), jnp.float32),
        scratch_types=[
            pltpu.VMEM((NCH, CHUNK), jnp.int32),
            pltpu.VMEM((NCH, CHUNK), jnp.int32),
            pltpu.VMEM((BPW,), jnp.float32),
            pltpu.VMEM((BPW,), jnp.float32),
            pltpu.VMEM((L,), jnp.float32),
            pltpu.VMEM((BPW,), jnp.float32),
            pltpu.SemaphoreType.DMA,
        ],
    )
    partial = bias_kern(uid, iid, buT, biT, b16)

    main_kern = pl.kernel(
        _mf_body,
        mesh=mesh,
        compiler_params=pltpu.CompilerParams(needs_layout_passes=False,
                                             use_tc_tiling_on_sc=True),
        out_type=jax.ShapeDtypeStruct((B,), jnp.float32),
        scratch_types=[
            pltpu.VMEM((NCH, CHUNK), jnp.int32),
            pltpu.VMEM((NCH, CHUNK), jnp.int32),
            pltpu.VMEM((2, CHUNK, D), jnp.float32),
            pltpu.VMEM((2, CHUNK, D), jnp.float32),
            pltpu.VMEM((BPW,), jnp.float32),
            pltpu.VMEM((D,), jnp.float32),
            pltpu.VMEM((L, L), jnp.float32),
            pltpu.VMEM((BPW,), jnp.float32),
            pltpu.SemaphoreType.DMA,
            pltpu.SemaphoreType.DMA,
        ],
    )
    return main_kern(uid, iid, user_table, item_table, w, partial)


def kernel(user_review, item_review, user_id, item_id, item_id_per_review,
           user_id_per_review, user_table, item_table, W, b, bu_table,
           bi_table):
    uid = user_id.reshape(NW * NCH, CHUNK)
    iid = item_id.reshape(NW * NCH, CHUNK)
    w = W.reshape(D)
    b16 = jnp.broadcast_to(b, (L,))
    return _mf_sc(uid, iid, user_table, item_table, w, b16,
                  bu_table.T, bi_table.T)
